# baseline (speedup 1.0000x reference)
import functools

import jax
import jax.numpy as jnp
from jax import lax
from jax.experimental import pallas as pl
from jax.experimental.pallas import tpu as pltpu
from jax.experimental.pallas import tpu_sc as plsc

_N = 10000
_E = 320000
_H = 128
_ED = 16

_NC = 2
_NS = 16
_NW = _NC * _NS
_NBUF = 4

_C = 80
_BE = 2000

_f32 = jnp.float32


def _sc_gather(table, dst3, srcn3, ne, c):
    epw = ne // _NW
    nch = epw // c
    mesh = plsc.VectorSubcoreMesh(core_axis_name="c", subcore_axis_name="s")

    @functools.partial(
        pl.kernel,
        mesh=mesh,
        out_type=jax.ShapeDtypeStruct((ne, _H), _f32),
        scratch_types=[
            pltpu.VMEM((nch, 1, c), jnp.int32),
            pltpu.VMEM((nch, 1, c), jnp.int32),
            pltpu.VMEM((_NBUF, c, _H), _f32),
            pltpu.VMEM((_NBUF, c, _H), _f32),
        ] + [pltpu.SemaphoreType.DMA] * (3 * _NBUF),
    )
    def k(t_hbm, dst_hbm, srcn_hbm, g_hbm, di, si, pbuf, qbuf, *sems):
        gp = sems[0:_NBUF]
        gq = sems[_NBUF:2 * _NBUF]
        wp = sems[2 * _NBUF:3 * _NBUF]
        wid = lax.axis_index("s") * _NC + lax.axis_index("c")
        pltpu.sync_copy(dst_hbm.at[wid], di)
        pltpu.sync_copy(srcn_hbm.at[wid], si)

        def issue_gather(j, b):
            pltpu.async_copy(t_hbm.at[di.at[j, 0]], pbuf.at[b], gp[b])
            pltpu.async_copy(t_hbm.at[si.at[j, 0]], qbuf.at[b], gq[b])

        def rows(j):
            return pl.ds(wid * epw + j * c, c)

        issue_gather(0, 0)
        issue_gather(1, 1)

        def step(i, carry):
            for b in range(_NBUF):
                j = i * _NBUF + b
                ba = (b + 2) % _NBUF

                @pl.when((j >= 2) & (j < nch + 2))
                def _():
                    pltpu.make_async_copy(pbuf.at[ba], g_hbm.at[rows(j - 2)],
                                          wp[ba]).wait()

                @pl.when(j + 2 < nch)
                def _():
                    issue_gather(j + 2, ba)

                @pl.when(j < nch)
                def _():
                    pltpu.make_async_copy(t_hbm.at[di.at[j, 0]], pbuf.at[b],
                                          gp[b]).wait()
                    pltpu.make_async_copy(t_hbm.at[si.at[j, 0]], qbuf.at[b],
                                          gq[b]).wait()

                    def row(r, rc):
                        for kk in range(_H // 16):
                            sl = pl.ds(kk * 16, 16)
                            pbuf[b, r, sl] = pbuf[b, r, sl] + qbuf[b, r, sl]
                        return rc

                    lax.fori_loop(0, c, row, 0)
                    pltpu.async_copy(pbuf.at[b], g_hbm.at[rows(j)], wp[b])
            return carry

        lax.fori_loop(0, (nch + 2 + _NBUF - 1) // _NBUF, step, 0)

    return k(table, dst3, srcn3)


def _sc_scatter(mg, dst3, zeros, ne, c):
    epw = ne // _NW
    nch = epw // c
    mesh = plsc.VectorSubcoreMesh(core_axis_name="c", subcore_axis_name="s")

    nbuf = 4

    @functools.partial(
        pl.kernel,
        mesh=mesh,
        out_type=jax.ShapeDtypeStruct((2 * _N, _H), _f32),
        scratch_types=[
            pltpu.VMEM_SHARED((_N, _H), _f32),
            pltpu.VMEM((nbuf, 1, c), jnp.int32),
            pltpu.VMEM((nbuf, c, _H), _f32),
        ] + [pltpu.SemaphoreType.DMA] * (3 * nbuf),
    )
    def k(mg_hbm, dst_hbm, z_hbm, out_hbm, acc_sh, ibuf, mbuf, *sems):
        rd = sems[0:nbuf]
        ri = sems[nbuf:2 * nbuf]
        sc = sems[2 * nbuf:3 * nbuf]
        cc = lax.axis_index("c")
        s = lax.axis_index("s")
        wid = s * _NC + cc

        @pl.when(s < 10)
        def _():
            pltpu.sync_copy(z_hbm.at[pl.ds(s * 1000, 1000)],
                            acc_sh.at[pl.ds(s * 1000, 1000)])

        plsc.subcore_barrier()

        def rows(j):
            return pl.ds(wid * epw + j * c, c)

        def issue_read(j, b):
            pltpu.async_copy(dst_hbm.at[wid, j], ibuf.at[b], ri[b])
            pltpu.async_copy(mg_hbm.at[rows(j)], mbuf.at[b], rd[b])

        issue_read(0, 0)
        issue_read(1, 1)

        def step(i, carry):
            for b in range(nbuf):
                j = i * nbuf + b
                ba = (b + 2) % nbuf

                @pl.when((j >= 2) & (j < nch + 2))
                def _():
                    pltpu.make_async_copy(mbuf.at[ba],
                                          acc_sh.at[ibuf.at[ba, 0]],
                                          sc[ba]).wait()

                @pl.when(j + 2 < nch)
                def _():
                    issue_read(j + 2, ba)

                @pl.when(j < nch)
                def _():
                    pltpu.make_async_copy(mg_hbm.at[rows(j)], mbuf.at[b],
                                          rd[b]).wait()
                    pltpu.make_async_copy(dst_hbm.at[wid, j], ibuf.at[b],
                                          ri[b]).wait()
                    pltpu.async_copy(mbuf.at[b], acc_sh.at[ibuf.at[b, 0]],
                                     sc[b], add=True)
            return carry

        lax.fori_loop(0, (nch + 2 + nbuf - 1) // nbuf, step, 0)
        plsc.subcore_barrier()

        @pl.when(s < 10)
        def _():
            pltpu.sync_copy(acc_sh.at[pl.ds(s * 1000, 1000)],
                            out_hbm.at[pl.ds(cc * _N + s * 1000, 1000)])

    return k(mg, dst3, zeros)


def _tc_project(h, w_stack):
    bn = 1000

    def body(h_ref, w_ref, o_ref):
        o_ref[...] = jnp.dot(h_ref[...], w_ref[0],
                             preferred_element_type=_f32)

    return pl.pallas_call(
        body,
        grid=(2 * _N // bn,),
        in_specs=[
            pl.BlockSpec((bn, _H), lambda g: (g % (_N // bn), 0)),
            pl.BlockSpec((1, _H, _H), lambda g: (g // (_N // bn), 0, 0)),
        ],
        out_specs=pl.BlockSpec((bn, _H), lambda g: (g, 0)),
        out_shape=jax.ShapeDtypeStruct((2 * _N, _H), _f32),
    )(h, w_stack)


def _tc_edge_mlp(attr, g, wa, b1, w2, b2, iw, ib, ne, be):
    def body(a_ref, g_ref, wa_ref, b1_ref, w2_ref, b2_ref, iw_ref,
             ib_ref, o_ref):
        x = (jnp.dot(a_ref[...], wa_ref[...], preferred_element_type=_f32)
             + g_ref[...] + b1_ref[...])
        m = jnp.maximum(x, 0.0)
        mij = jnp.maximum(
            jnp.dot(m, w2_ref[...], preferred_element_type=_f32)
            + b2_ref[...], 0.0)
        t = jnp.sum(mij * iw_ref[...], axis=1, keepdims=True) + ib_ref[0, 0]
        eij = 1.0 / (1.0 + jnp.exp(-t))
        o_ref[...] = mij * eij

    full = lambda g: (0, 0)
    return pl.pallas_call(
        body,
        grid=(ne // be,),
        in_specs=[
            pl.BlockSpec((be, _ED), lambda g: (g, 0)),
            pl.BlockSpec((be, _H), lambda g: (g, 0)),
            pl.BlockSpec((_ED, _H), full),
            pl.BlockSpec((1, _H), full),
            pl.BlockSpec((_H, _H), full),
            pl.BlockSpec((1, _H), full),
            pl.BlockSpec((1, _H), full),
            pl.BlockSpec((1, 1), full),
        ],
        out_specs=pl.BlockSpec((be, _H), lambda g: (g, 0)),
        out_shape=jax.ShapeDtypeStruct((ne, _H), _f32),
    )(attr, g, wa, b1, w2, b2, iw, ib)


def _tc_node_mlp(partials, h, wmi, wh, b1, w2, b2):
    bn = 1000

    def body(p0_ref, p1_ref, h_ref, wmi_ref, wh_ref, b1_ref, w2_ref, b2_ref,
             o_ref):
        mi = p0_ref[...] + p1_ref[...]
        z = jnp.maximum(
            jnp.dot(mi, wmi_ref[...], preferred_element_type=_f32)
            + jnp.dot(h_ref[...], wh_ref[...], preferred_element_type=_f32)
            + b1_ref[...], 0.0)
        o_ref[...] = jnp.dot(z, w2_ref[...],
                             preferred_element_type=_f32) + b2_ref[...]

    full = lambda g: (0, 0)
    nb = _N // bn
    return pl.pallas_call(
        body,
        grid=(nb,),
        in_specs=[
            pl.BlockSpec((bn, _H), lambda g: (g, 0)),
            pl.BlockSpec((bn, _H), lambda g: (g + nb, 0)),
            pl.BlockSpec((bn, _H), lambda g: (g, 0)),
            pl.BlockSpec((_H, _H), full),
            pl.BlockSpec((_H, _H), full),
            pl.BlockSpec((1, _H), full),
            pl.BlockSpec((_H, _H), full),
            pl.BlockSpec((1, _H), full),
        ],
        out_specs=pl.BlockSpec((bn, _H), lambda g: (g, 0)),
        out_shape=jax.ShapeDtypeStruct((_N, _H), _f32),
    )(partials, partials, h, wmi, wh, b1, w2, b2)


def kernel(h, edge_index, edge_attr, e_w1, e_b1, e_w2, e_b2, i_w, i_b,
           n_w1, n_b1, n_w2, n_b2):
    dst = edge_index[0].astype(jnp.int32)
    srcn = (edge_index[1] + _N).astype(jnp.int32)

    w_stack = jnp.stack([e_w1[_ED:_ED + _H], e_w1[_ED + _H:]])
    table = _tc_project(h, w_stack)

    zeros = jnp.zeros((_N, _H), _f32)
    wa = e_w1[:_ED]
    b1 = e_b1.reshape(1, _H)
    b2 = e_b2.reshape(1, _H)
    iw = i_w.reshape(1, _H)
    ib = i_b.reshape(1, 1)

    epw = _E // _NW
    d3 = dst.reshape(_NW, epw // _C, 1, _C)
    s3 = srcn.reshape(_NW, epw // _C, 1, _C)

    g = _sc_gather(table, d3, s3, _E, _C)
    mg = _tc_edge_mlp(edge_attr, g, wa, b1, e_w2, b2, iw, ib, _E, _BE)
    partials = _sc_scatter(mg, d3, zeros, _E, _C)

    return _tc_node_mlp(partials, h,
                        n_w1[:_H], n_w1[_H:], n_b1.reshape(1, _H),
                        n_w2, n_b2.reshape(1, _H))


# speedup vs baseline: 1.0175x; 1.0175x over previous
import functools

import jax
import jax.numpy as jnp
from jax import lax
from jax.experimental import pallas as pl
from jax.experimental.pallas import tpu as pltpu
from jax.experimental.pallas import tpu_sc as plsc

_N = 10000
_E = 320000
_H = 128
_ED = 16

_NC = 2
_NS = 16
_NW = _NC * _NS
_NBUF = 4

_SPLITS = ((0, 107520, 112, 1920),
           (107520, 107520, 112, 1920),
           (215040, 104960, 80, 1640))
_SCC = 80

_f32 = jnp.float32


def _sc_gather(table, dst3, srcn3, ne, c):
    epw = ne // _NW
    nch = epw // c
    mesh = plsc.VectorSubcoreMesh(core_axis_name="c", subcore_axis_name="s")

    @functools.partial(
        pl.kernel,
        mesh=mesh,
        out_type=jax.ShapeDtypeStruct((ne, _H), _f32),
        scratch_types=[
            pltpu.VMEM((nch, 1, c), jnp.int32),
            pltpu.VMEM((nch, 1, c), jnp.int32),
            pltpu.VMEM((_NBUF, c, _H), _f32),
            pltpu.VMEM((_NBUF, c, _H), _f32),
        ] + [pltpu.SemaphoreType.DMA] * (3 * _NBUF),
    )
    def k(t_hbm, dst_hbm, srcn_hbm, g_hbm, di, si, pbuf, qbuf, *sems):
        gp = sems[0:_NBUF]
        gq = sems[_NBUF:2 * _NBUF]
        wp = sems[2 * _NBUF:3 * _NBUF]
        wid = lax.axis_index("s") * _NC + lax.axis_index("c")
        pltpu.sync_copy(dst_hbm.at[wid], di)
        pltpu.sync_copy(srcn_hbm.at[wid], si)

        def issue_gather(j, b):
            pltpu.async_copy(t_hbm.at[di.at[j, 0]], pbuf.at[b], gp[b])
            pltpu.async_copy(t_hbm.at[si.at[j, 0]], qbuf.at[b], gq[b])

        def rows(j):
            return pl.ds(wid * epw + j * c, c)

        issue_gather(0, 0)
        issue_gather(1, 1)

        def step(i, carry):
            for b in range(_NBUF):
                j = i * _NBUF + b
                ba = (b + 2) % _NBUF

                @pl.when((j >= 2) & (j < nch + 2))
                def _():
                    pltpu.make_async_copy(pbuf.at[ba], g_hbm.at[rows(j - 2)],
                                          wp[ba]).wait()

                @pl.when(j + 2 < nch)
                def _():
                    issue_gather(j + 2, ba)

                @pl.when(j < nch)
                def _():
                    pltpu.make_async_copy(t_hbm.at[di.at[j, 0]], pbuf.at[b],
                                          gp[b]).wait()
                    pltpu.make_async_copy(t_hbm.at[si.at[j, 0]], qbuf.at[b],
                                          gq[b]).wait()

                    def row(r, rc):
                        for kk in range(_H // 16):
                            sl = pl.ds(kk * 16, 16)
                            pbuf[b, r, sl] = pbuf[b, r, sl] + qbuf[b, r, sl]
                        return rc

                    lax.fori_loop(0, c, row, 0)
                    pltpu.async_copy(pbuf.at[b], g_hbm.at[rows(j)], wp[b])
            return carry

        lax.fori_loop(0, (nch + 2 + _NBUF - 1) // _NBUF, step, 0)

    return k(table, dst3, srcn3)


def _sc_scatter(mg, dst3, zeros, ne, c):
    epw = ne // _NW
    nch = epw // c
    mesh = plsc.VectorSubcoreMesh(core_axis_name="c", subcore_axis_name="s")

    nbuf = 4

    @functools.partial(
        pl.kernel,
        mesh=mesh,
        out_type=jax.ShapeDtypeStruct((2 * _N, _H), _f32),
        scratch_types=[
            pltpu.VMEM_SHARED((_N, _H), _f32),
            pltpu.VMEM((nbuf, 1, c), jnp.int32),
            pltpu.VMEM((nbuf, c, _H), _f32),
        ] + [pltpu.SemaphoreType.DMA] * (3 * nbuf),
    )
    def k(mg_hbm, dst_hbm, z_hbm, out_hbm, acc_sh, ibuf, mbuf, *sems):
        rd = sems[0:nbuf]
        ri = sems[nbuf:2 * nbuf]
        sc = sems[2 * nbuf:3 * nbuf]
        cc = lax.axis_index("c")
        s = lax.axis_index("s")
        wid = s * _NC + cc

        @pl.when(s < 10)
        def _():
            pltpu.sync_copy(z_hbm.at[pl.ds(s * 1000, 1000)],
                            acc_sh.at[pl.ds(s * 1000, 1000)])

        plsc.subcore_barrier()

        def rows(j):
            return pl.ds(wid * epw + j * c, c)

        def issue_read(j, b):
            pltpu.async_copy(dst_hbm.at[wid, j], ibuf.at[b], ri[b])
            pltpu.async_copy(mg_hbm.at[rows(j)], mbuf.at[b], rd[b])

        issue_read(0, 0)
        issue_read(1, 1)

        def step(i, carry):
            for b in range(nbuf):
                j = i * nbuf + b
                ba = (b + 2) % nbuf

                @pl.when((j >= 2) & (j < nch + 2))
                def _():
                    pltpu.make_async_copy(mbuf.at[ba],
                                          acc_sh.at[ibuf.at[ba, 0]],
                                          sc[ba]).wait()

                @pl.when(j + 2 < nch)
                def _():
                    issue_read(j + 2, ba)

                @pl.when(j < nch)
                def _():
                    pltpu.make_async_copy(mg_hbm.at[rows(j)], mbuf.at[b],
                                          rd[b]).wait()
                    pltpu.make_async_copy(dst_hbm.at[wid, j], ibuf.at[b],
                                          ri[b]).wait()
                    pltpu.async_copy(mbuf.at[b], acc_sh.at[ibuf.at[b, 0]],
                                     sc[b], add=True)
            return carry

        lax.fori_loop(0, (nch + 2 + nbuf - 1) // nbuf, step, 0)
        plsc.subcore_barrier()

        @pl.when(s < 10)
        def _():
            pltpu.sync_copy(acc_sh.at[pl.ds(s * 1000, 1000)],
                            out_hbm.at[pl.ds(cc * _N + s * 1000, 1000)])

    return k(mg, dst3, zeros)


def _tc_project(h, w_stack):
    bn = 1000

    def body(h_ref, w_ref, o_ref):
        o_ref[...] = jnp.dot(h_ref[...], w_ref[0],
                             preferred_element_type=_f32)

    return pl.pallas_call(
        body,
        grid=(2 * _N // bn,),
        in_specs=[
            pl.BlockSpec((bn, _H), lambda g: (g % (_N // bn), 0)),
            pl.BlockSpec((1, _H, _H), lambda g: (g // (_N // bn), 0, 0)),
        ],
        out_specs=pl.BlockSpec((bn, _H), lambda g: (g, 0)),
        out_shape=jax.ShapeDtypeStruct((2 * _N, _H), _f32),
    )(h, w_stack)


def _tc_edge_mlp(attr, g, wa, b1, w2, b2, iw, ib, ne, be):
    def body(a_ref, g_ref, wa_ref, b1_ref, w2_ref, b2_ref, iw_ref,
             ib_ref, o_ref):
        x = (jnp.dot(a_ref[...], wa_ref[...], preferred_element_type=_f32)
             + g_ref[...] + b1_ref[...])
        m = jnp.maximum(x, 0.0)
        mij = jnp.maximum(
            jnp.dot(m, w2_ref[...], preferred_element_type=_f32)
            + b2_ref[...], 0.0)
        t = jnp.sum(mij * iw_ref[...], axis=1, keepdims=True) + ib_ref[0, 0]
        eij = 1.0 / (1.0 + jnp.exp(-t))
        o_ref[...] = mij * eij

    full = lambda g: (0, 0)
    return pl.pallas_call(
        body,
        grid=(ne // be,),
        in_specs=[
            pl.BlockSpec((be, _ED), lambda g: (g, 0)),
            pl.BlockSpec((be, _H), lambda g: (g, 0)),
            pl.BlockSpec((_ED, _H), full),
            pl.BlockSpec((1, _H), full),
            pl.BlockSpec((_H, _H), full),
            pl.BlockSpec((1, _H), full),
            pl.BlockSpec((1, _H), full),
            pl.BlockSpec((1, 1), full),
        ],
        out_specs=pl.BlockSpec((be, _H), lambda g: (g, 0)),
        out_shape=jax.ShapeDtypeStruct((ne, _H), _f32),
    )(attr, g, wa, b1, w2, b2, iw, ib)


def _tc_node_mlp(parts, h, wmi, wh, b1, w2, b2):
    bn = 1000

    def body(pa0, pa1, pb0, pb1, pc0, pc1, h_ref, wmi_ref, wh_ref,
             b1_ref, w2_ref, b2_ref, o_ref):
        mi = ((pa0[...] + pa1[...]) + (pb0[...] + pb1[...])
              + (pc0[...] + pc1[...]))
        z = jnp.maximum(
            jnp.dot(mi, wmi_ref[...], preferred_element_type=_f32)
            + jnp.dot(h_ref[...], wh_ref[...], preferred_element_type=_f32)
            + b1_ref[...], 0.0)
        o_ref[...] = jnp.dot(z, w2_ref[...],
                             preferred_element_type=_f32) + b2_ref[...]

    full = lambda g: (0, 0)
    nb = _N // bn
    lo_spec = pl.BlockSpec((bn, _H), lambda g: (g, 0))
    hi_spec = pl.BlockSpec((bn, _H), lambda g: (g + nb, 0))
    return pl.pallas_call(
        body,
        grid=(nb,),
        in_specs=[
            lo_spec, hi_spec, lo_spec, hi_spec, lo_spec, hi_spec,
            lo_spec,
            pl.BlockSpec((_H, _H), full),
            pl.BlockSpec((_H, _H), full),
            pl.BlockSpec((1, _H), full),
            pl.BlockSpec((_H, _H), full),
            pl.BlockSpec((1, _H), full),
        ],
        out_specs=pl.BlockSpec((bn, _H), lambda g: (g, 0)),
        out_shape=jax.ShapeDtypeStruct((_N, _H), _f32),
    )(parts[0], parts[0], parts[1], parts[1], parts[2], parts[2],
      h, wmi, wh, b1, w2, b2)


def kernel(h, edge_index, edge_attr, e_w1, e_b1, e_w2, e_b2, i_w, i_b,
           n_w1, n_b1, n_w2, n_b2):
    dst = edge_index[0].astype(jnp.int32)
    srcn = (edge_index[1] + _N).astype(jnp.int32)

    w_stack = jnp.stack([e_w1[_ED:_ED + _H], e_w1[_ED + _H:]])
    table = _tc_project(h, w_stack)

    zeros = jnp.zeros((_N, _H), _f32)
    wa = e_w1[:_ED]
    b1 = e_b1.reshape(1, _H)
    b2 = e_b2.reshape(1, _H)
    iw = i_w.reshape(1, _H)
    ib = i_b.reshape(1, 1)

    slices = []
    for lo, ne, gc, be in _SPLITS:
        epw = ne // _NW
        dh = lax.dynamic_slice_in_dim(dst, lo, ne)
        sh = lax.dynamic_slice_in_dim(srcn, lo, ne)
        slices.append({
            "lo": lo, "ne": ne, "gc": gc, "be": be,
            "gd3": dh.reshape(_NW, epw // gc, 1, gc),
            "gs3": sh.reshape(_NW, epw // gc, 1, gc),
            "sd3": dh.reshape(_NW, epw // _SCC, 1, _SCC),
        })

    for sl in slices:
        sl["g"] = _sc_gather(table, sl["gd3"], sl["gs3"], sl["ne"], sl["gc"])

    for sl in slices:
        attr = lax.dynamic_slice_in_dim(edge_attr, sl["lo"], sl["ne"])
        sl["mg"] = _tc_edge_mlp(attr, sl["g"], wa, b1, e_w2, b2, iw, ib,
                                sl["ne"], sl["be"])

    parts = [_sc_scatter(sl["mg"], sl["sd3"], zeros, sl["ne"], _SCC)
             for sl in slices]

    return _tc_node_mlp(parts, h,
                        n_w1[:_H], n_w1[_H:], n_b1.reshape(1, _H),
                        n_w2, n_b2.reshape(1, _H))


# speedup vs baseline: 1.0360x; 1.0182x over previous
import functools

import jax
import jax.numpy as jnp
from jax import lax
from jax.experimental import pallas as pl
from jax.experimental.pallas import tpu as pltpu
from jax.experimental.pallas import tpu_sc as plsc

_N = 10000
_E = 320000
_H = 128
_ED = 16

_NC = 2
_NS = 16
_NW = _NC * _NS
_NBUF = 4

_SPLITS = ((0, 161280, 112, 2016),
           (161280, 158720, 80, 2480))
_SCC = 80

_f32 = jnp.float32


def _sc_gather(table, dst3, srcn3, ne, c):
    epw = ne // _NW
    nch = epw // c
    mesh = plsc.VectorSubcoreMesh(core_axis_name="c", subcore_axis_name="s")

    @functools.partial(
        pl.kernel,
        mesh=mesh,
        out_type=jax.ShapeDtypeStruct((ne, _H), _f32),
        scratch_types=[
            pltpu.VMEM((nch, 1, c), jnp.int32),
            pltpu.VMEM((nch, 1, c), jnp.int32),
            pltpu.VMEM((_NBUF, c, _H), _f32),
            pltpu.VMEM((_NBUF, c, _H), _f32),
        ] + [pltpu.SemaphoreType.DMA] * (3 * _NBUF),
    )
    def k(t_hbm, dst_hbm, srcn_hbm, g_hbm, di, si, pbuf, qbuf, *sems):
        gp = sems[0:_NBUF]
        gq = sems[_NBUF:2 * _NBUF]
        wp = sems[2 * _NBUF:3 * _NBUF]
        wid = lax.axis_index("s") * _NC + lax.axis_index("c")
        pltpu.sync_copy(dst_hbm.at[wid], di)
        pltpu.sync_copy(srcn_hbm.at[wid], si)

        def issue_gather(j, b):
            pltpu.async_copy(t_hbm.at[di.at[j, 0]], pbuf.at[b], gp[b])
            pltpu.async_copy(t_hbm.at[si.at[j, 0]], qbuf.at[b], gq[b])

        def rows(j):
            return pl.ds(wid * epw + j * c, c)

        issue_gather(0, 0)
        issue_gather(1, 1)

        def step(i, carry):
            for b in range(_NBUF):
                j = i * _NBUF + b
                ba = (b + 2) % _NBUF

                @pl.when((j >= 2) & (j < nch + 2))
                def _():
                    pltpu.make_async_copy(pbuf.at[ba], g_hbm.at[rows(j - 2)],
                                          wp[ba]).wait()

                @pl.when(j + 2 < nch)
                def _():
                    issue_gather(j + 2, ba)

                @pl.when(j < nch)
                def _():
                    pltpu.make_async_copy(t_hbm.at[di.at[j, 0]], pbuf.at[b],
                                          gp[b]).wait()
                    pltpu.make_async_copy(t_hbm.at[si.at[j, 0]], qbuf.at[b],
                                          gq[b]).wait()

                    def row(r, rc):
                        for kk in range(_H // 16):
                            sl = pl.ds(kk * 16, 16)
                            pbuf[b, r, sl] = pbuf[b, r, sl] + qbuf[b, r, sl]
                        return rc

                    lax.fori_loop(0, c, row, 0)
                    pltpu.async_copy(pbuf.at[b], g_hbm.at[rows(j)], wp[b])
            return carry

        lax.fori_loop(0, (nch + 2 + _NBUF - 1) // _NBUF, step, 0)

    return k(table, dst3, srcn3)


def _sc_scatter(mg, dst3, zeros, ne, c):
    epw = ne // _NW
    nch = epw // c
    mesh = plsc.VectorSubcoreMesh(core_axis_name="c", subcore_axis_name="s")

    nbuf = 4

    @functools.partial(
        pl.kernel,
        mesh=mesh,
        out_type=jax.ShapeDtypeStruct((2 * _N, _H), _f32),
        scratch_types=[
            pltpu.VMEM_SHARED((_N, _H), _f32),
            pltpu.VMEM((nbuf, 1, c), jnp.int32),
            pltpu.VMEM((nbuf, c, _H), _f32),
        ] + [pltpu.SemaphoreType.DMA] * (3 * nbuf),
    )
    def k(mg_hbm, dst_hbm, z_hbm, out_hbm, acc_sh, ibuf, mbuf, *sems):
        rd = sems[0:nbuf]
        ri = sems[nbuf:2 * nbuf]
        sc = sems[2 * nbuf:3 * nbuf]
        cc = lax.axis_index("c")
        s = lax.axis_index("s")
        wid = s * _NC + cc

        @pl.when(s < 10)
        def _():
            pltpu.sync_copy(z_hbm.at[pl.ds(s * 1000, 1000)],
                            acc_sh.at[pl.ds(s * 1000, 1000)])

        plsc.subcore_barrier()

        def rows(j):
            return pl.ds(wid * epw + j * c, c)

        def issue_read(j, b):
            pltpu.async_copy(dst_hbm.at[wid, j], ibuf.at[b], ri[b])
            pltpu.async_copy(mg_hbm.at[rows(j)], mbuf.at[b], rd[b])

        issue_read(0, 0)
        issue_read(1, 1)

        def step(i, carry):
            for b in range(nbuf):
                j = i * nbuf + b
                ba = (b + 2) % nbuf

                @pl.when((j >= 2) & (j < nch + 2))
                def _():
                    pltpu.make_async_copy(mbuf.at[ba],
                                          acc_sh.at[ibuf.at[ba, 0]],
                                          sc[ba]).wait()

                @pl.when(j + 2 < nch)
                def _():
                    issue_read(j + 2, ba)

                @pl.when(j < nch)
                def _():
                    pltpu.make_async_copy(mg_hbm.at[rows(j)], mbuf.at[b],
                                          rd[b]).wait()
                    pltpu.make_async_copy(dst_hbm.at[wid, j], ibuf.at[b],
                                          ri[b]).wait()
                    pltpu.async_copy(mbuf.at[b], acc_sh.at[ibuf.at[b, 0]],
                                     sc[b], add=True)
            return carry

        lax.fori_loop(0, (nch + 2 + nbuf - 1) // nbuf, step, 0)
        plsc.subcore_barrier()

        @pl.when(s < 10)
        def _():
            pltpu.sync_copy(acc_sh.at[pl.ds(s * 1000, 1000)],
                            out_hbm.at[pl.ds(cc * _N + s * 1000, 1000)])

    return k(mg, dst3, zeros)


def _tc_project(h, w_stack):
    bn = 1000

    def body(h_ref, w_ref, o_ref):
        o_ref[...] = jnp.dot(h_ref[...], w_ref[0],
                             preferred_element_type=_f32)

    return pl.pallas_call(
        body,
        grid=(2 * _N // bn,),
        in_specs=[
            pl.BlockSpec((bn, _H), lambda g: (g % (_N // bn), 0)),
            pl.BlockSpec((1, _H, _H), lambda g: (g // (_N // bn), 0, 0)),
        ],
        out_specs=pl.BlockSpec((bn, _H), lambda g: (g, 0)),
        out_shape=jax.ShapeDtypeStruct((2 * _N, _H), _f32),
    )(h, w_stack)


def _tc_edge_mlp(attr, g, wa, b1, w2, b2, iw, ib, ne, be):
    def body(a_ref, g_ref, wa_ref, b1_ref, w2_ref, b2_ref, iw_ref,
             ib_ref, o_ref):
        x = (jnp.dot(a_ref[...], wa_ref[...], preferred_element_type=_f32)
             + g_ref[...] + b1_ref[...])
        m = jnp.maximum(x, 0.0)
        mij = jnp.maximum(
            jnp.dot(m, w2_ref[...], preferred_element_type=_f32)
            + b2_ref[...], 0.0)
        t = jnp.sum(mij * iw_ref[...], axis=1, keepdims=True) + ib_ref[0, 0]
        eij = 1.0 / (1.0 + jnp.exp(-t))
        o_ref[...] = mij * eij

    full = lambda g: (0, 0)
    return pl.pallas_call(
        body,
        grid=(ne // be,),
        in_specs=[
            pl.BlockSpec((be, _ED), lambda g: (g, 0)),
            pl.BlockSpec((be, _H), lambda g: (g, 0)),
            pl.BlockSpec((_ED, _H), full),
            pl.BlockSpec((1, _H), full),
            pl.BlockSpec((_H, _H), full),
            pl.BlockSpec((1, _H), full),
            pl.BlockSpec((1, _H), full),
            pl.BlockSpec((1, 1), full),
        ],
        out_specs=pl.BlockSpec((be, _H), lambda g: (g, 0)),
        out_shape=jax.ShapeDtypeStruct((ne, _H), _f32),
    )(attr, g, wa, b1, w2, b2, iw, ib)


def _tc_node_mlp(parts, h, wmi, wh, b1, w2, b2):
    bn = 1000

    def body(pa0, pa1, pb0, pb1, h_ref, wmi_ref, wh_ref,
             b1_ref, w2_ref, b2_ref, o_ref):
        mi = (pa0[...] + pa1[...]) + (pb0[...] + pb1[...])
        z = jnp.maximum(
            jnp.dot(mi, wmi_ref[...], preferred_element_type=_f32)
            + jnp.dot(h_ref[...], wh_ref[...], preferred_element_type=_f32)
            + b1_ref[...], 0.0)
        o_ref[...] = jnp.dot(z, w2_ref[...],
                             preferred_element_type=_f32) + b2_ref[...]

    full = lambda g: (0, 0)
    nb = _N // bn
    lo_spec = pl.BlockSpec((bn, _H), lambda g: (g, 0))
    hi_spec = pl.BlockSpec((bn, _H), lambda g: (g + nb, 0))
    return pl.pallas_call(
        body,
        grid=(nb,),
        in_specs=[
            lo_spec, hi_spec, lo_spec, hi_spec,
            lo_spec,
            pl.BlockSpec((_H, _H), full),
            pl.BlockSpec((_H, _H), full),
            pl.BlockSpec((1, _H), full),
            pl.BlockSpec((_H, _H), full),
            pl.BlockSpec((1, _H), full),
        ],
        out_specs=pl.BlockSpec((bn, _H), lambda g: (g, 0)),
        out_shape=jax.ShapeDtypeStruct((_N, _H), _f32),
    )(parts[0], parts[0], parts[1], parts[1],
      h, wmi, wh, b1, w2, b2)


def kernel(h, edge_index, edge_attr, e_w1, e_b1, e_w2, e_b2, i_w, i_b,
           n_w1, n_b1, n_w2, n_b2):
    dst = edge_index[0].astype(jnp.int32)
    srcn = (edge_index[1] + _N).astype(jnp.int32)

    w_stack = jnp.stack([e_w1[_ED:_ED + _H], e_w1[_ED + _H:]])
    table = _tc_project(h, w_stack)

    zeros = jnp.zeros((_N, _H), _f32)
    wa = e_w1[:_ED]
    b1 = e_b1.reshape(1, _H)
    b2 = e_b2.reshape(1, _H)
    iw = i_w.reshape(1, _H)
    ib = i_b.reshape(1, 1)

    slices = []
    for lo, ne, gc, be in _SPLITS:
        epw = ne // _NW
        dh = lax.dynamic_slice_in_dim(dst, lo, ne)
        sh = lax.dynamic_slice_in_dim(srcn, lo, ne)
        slices.append({
            "lo": lo, "ne": ne, "gc": gc, "be": be,
            "gd3": dh.reshape(_NW, epw // gc, 1, gc),
            "gs3": sh.reshape(_NW, epw // gc, 1, gc),
            "sd3": dh.reshape(_NW, epw // _SCC, 1, _SCC),
        })

    for sl in slices:
        sl["g"] = _sc_gather(table, sl["gd3"], sl["gs3"], sl["ne"], sl["gc"])

    for sl in slices:
        attr = lax.dynamic_slice_in_dim(edge_attr, sl["lo"], sl["ne"])
        sl["mg"] = _tc_edge_mlp(attr, sl["g"], wa, b1, e_w2, b2, iw, ib,
                                sl["ne"], sl["be"])

    parts = [_sc_scatter(sl["mg"], sl["sd3"], zeros, sl["ne"], _SCC)
             for sl in slices]

    return _tc_node_mlp(parts, h,
                        n_w1[:_H], n_w1[_H:], n_b1.reshape(1, _H),
                        n_w2, n_b2.reshape(1, _H))
